# Initial kernel scaffold; baseline (speedup 1.0000x reference)
import jax
import jax.numpy as jnp
from jax.experimental import pallas as pl


def kernel(x, edge_attr, edge_index, batch, params):
    raise NotImplementedError("write your pallas kernel here")


# speedup vs baseline: 2.3370x; 2.3370x over previous
import functools

import jax
import jax.numpy as jnp
from jax import lax
from jax.experimental import pallas as pl
from jax.experimental.pallas import tpu as pltpu
from jax.experimental.pallas import tpu_sc as plsc

_N = 10000
_E = 320000
_G = 64
_H = 128
_L = 3

_NC = 2
_NS = 16
_EPW = _E // (_NC * _NS)
_C = 40
_NCH = _EPW // _C
_CPG = 50
_GRP = _NCH // _CPG
_RPS = 624
_ZCH = 16


def _embed_body(x_ref, w_ref, b_ref, o_ref):
    o_ref[...] = (
        jnp.dot(x_ref[...], w_ref[...], preferred_element_type=jnp.float32)
        + b_ref[...]
    )


def _embed(x, W, b):
    return pl.pallas_call(
        _embed_body,
        out_shape=jax.ShapeDtypeStruct((_N, _H), jnp.float32),
    )(x, W, b.reshape(1, _H))


_RB = 4000


def _edge_e_body(ea_ref, w_ref, b_ref, o_ref):
    o_ref[0] = (
        jnp.dot(ea_ref[...], w_ref[0], preferred_element_type=jnp.float32)
        + b_ref[0]
    )


def _edge_e(edge_attr, EW, EB):
    return pl.pallas_call(
        _edge_e_body,
        grid=(_L, _E // _RB),
        in_specs=[
            pl.BlockSpec((_RB, 6), lambda l, r: (r, 0)),
            pl.BlockSpec((1, 6, _H), lambda l, r: (l, 0, 0)),
            pl.BlockSpec((1, 1, _H), lambda l, r: (l, 0, 0)),
        ],
        out_specs=pl.BlockSpec((1, _RB, _H), lambda l, r: (l, r, 0)),
        out_shape=jax.ShapeDtypeStruct((_L, _E, _H), jnp.float32),
    )(edge_attr, EW, EB)


_MB = 2000


def _mlp_body(s_ref, h_ref, a_ref, w1_ref, b1_ref, w2_ref, b2_ref, o_ref):
    z = s_ref[0] * h_ref[...] + a_ref[0] + a_ref[1]
    z = jnp.maximum(
        jnp.dot(z, w1_ref[...], preferred_element_type=jnp.float32) + b1_ref[...],
        0.0,
    )
    z = jnp.dot(z, w2_ref[...], preferred_element_type=jnp.float32) + b2_ref[...]
    o_ref[...] = jnp.maximum(z, 0.0)


def _mlp(h, agg, scale, W1, b1, W2, b2):
    return pl.pallas_call(
        _mlp_body,
        grid=(_N // _MB,),
        in_specs=[
            pl.BlockSpec(memory_space=pltpu.SMEM),
            pl.BlockSpec((_MB, _H), lambda r: (r, 0)),
            pl.BlockSpec((_NC, _MB, _H), lambda r: (0, r, 0)),
            pl.BlockSpec((_H, 2 * _H), lambda r: (0, 0)),
            pl.BlockSpec((1, 2 * _H), lambda r: (0, 0)),
            pl.BlockSpec((2 * _H, _H), lambda r: (0, 0)),
            pl.BlockSpec((1, _H), lambda r: (0, 0)),
        ],
        out_specs=pl.BlockSpec((_MB, _H), lambda r: (r, 0)),
        out_shape=jax.ShapeDtypeStruct((_N, _H), jnp.float32),
    )(scale, h, agg, W1, b1.reshape(1, 2 * _H), W2, b2.reshape(1, _H))


def _pool_body(h_ref, bat_ref, w1_ref, b1_ref, w2_ref, b2_ref, o_ref):
    ids = bat_ref[...]
    oh = (ids == lax.broadcasted_iota(jnp.int32, (1, _G), 1)).astype(jnp.float32)
    summed = lax.dot_general(
        oh, h_ref[...], (((0,), (0,)), ((), ())),
        preferred_element_type=jnp.float32,
    )
    counts = lax.dot_general(
        oh, jnp.ones((_N, 1), jnp.float32), (((0,), (0,)), ((), ())),
        preferred_element_type=jnp.float32,
    )
    g = summed / jnp.clip(counts, 1.0, None)
    g = jnp.maximum(
        jnp.dot(g, w1_ref[...], preferred_element_type=jnp.float32) + b1_ref[...],
        0.0,
    )
    o_ref[...] = (
        jnp.dot(g, w2_ref[...], preferred_element_type=jnp.float32) + b2_ref[...]
    )


def _pool(h, batch_col, Wm1, bm1, Wm2, bm2):
    return pl.pallas_call(
        _pool_body,
        out_shape=jax.ShapeDtypeStruct((_G, 1), jnp.float32),
    )(h, batch_col, Wm1, bm1.reshape(1, _H // 2), Wm2, bm2.reshape(1, 1))


def _sc_edge(h, e_l, src4, dst4):
    mesh = plsc.VectorSubcoreMesh(core_axis_name="c", subcore_axis_name="s")

    @functools.partial(
        pl.kernel,
        out_type=jax.ShapeDtypeStruct((_NC, _N, _H), jnp.float32),
        mesh=mesh,
        scratch_types=[
            pltpu.VMEM((_CPG, _C), jnp.int32),
            pltpu.VMEM((_CPG, _C), jnp.int32),
            pltpu.VMEM((_C, _H), jnp.float32),
            pltpu.VMEM((_C, _H), jnp.float32),
            pltpu.VMEM((_ZCH, _H), jnp.float32),
            pltpu.VMEM_SHARED((_N, _H), jnp.float32),
            pltpu.SemaphoreType.DMA,
            pltpu.SemaphoreType.DMA,
        ],
    )
    def body(h_hbm, e_hbm, src_hbm, dst_hbm, out_hbm,
             srcv, dstv, hv, ev, zv, aggr, sem_g, sem_e):
        cid = lax.axis_index("c")
        sid = lax.axis_index("s")

        zero = jnp.zeros((16,), jnp.float32)

        def zrow(i, carry):
            for j in range(_H // 16):
                zv[i, pl.ds(j * 16, 16)] = zero
            return carry

        lax.fori_loop(0, _ZCH, zrow, 0)

        def zcopy(k, carry):
            pltpu.sync_copy(zv, aggr.at[pl.ds(sid * _RPS + k * _ZCH, _ZCH)])
            return carry

        lax.fori_loop(0, _RPS // _ZCH, zcopy, 0)

        @pl.when(sid == _NS - 1)
        def _zero_tail():
            pltpu.sync_copy(zv, aggr.at[pl.ds(_NS * _RPS, 16)])

        plsc.subcore_barrier()

        ebase = cid * (_E // _NC) + sid * _EPW

        def group(g, carry):
            pltpu.sync_copy(src_hbm.at[cid, sid, g], srcv)
            pltpu.sync_copy(dst_hbm.at[cid, sid, g], dstv)

            def chunk(k, c1):
                cp_g = pltpu.async_copy(h_hbm.at[srcv.at[k]], hv, sem_g)
                cp_e = pltpu.async_copy(
                    e_hbm.at[pl.ds(ebase + (g * _CPG + k) * _C, _C)], ev, sem_e)
                cp_g.wait()
                cp_e.wait()

                def row(i, c2):
                    for j in range(_H // 16):
                        s = pl.ds(j * 16, 16)
                        hv[i, s] = jnp.maximum(hv[i, s] + ev[i, s], 0.0)
                    return c2

                lax.fori_loop(0, _C, row, 0)
                pltpu.sync_copy(hv, aggr.at[dstv.at[k]], add=True)
                return c1

            lax.fori_loop(0, _CPG, chunk, 0)
            return carry

        lax.fori_loop(0, _GRP, group, 0)
        plsc.subcore_barrier()

        for k in range(3):
            pltpu.sync_copy(
                aggr.at[pl.ds(sid * _RPS + k * 208, 208)],
                out_hbm.at[cid, pl.ds(sid * _RPS + k * 208, 208)],
            )

        @pl.when(sid == _NS - 1)
        def _out_tail():
            pltpu.sync_copy(
                aggr.at[pl.ds(_NS * _RPS, 16)],
                out_hbm.at[cid, pl.ds(_NS * _RPS, 16)],
            )

    return body(h, e_l, src4, dst4)


def kernel(x, edge_attr, edge_index, batch, params):
    src4 = edge_index[0].reshape(_NC, _NS, _GRP, _CPG, _C)
    dst4 = edge_index[1].reshape(_NC, _NS, _GRP, _CPG, _C)

    layers = params["layers"]
    EW = jnp.stack([params["We"] @ layers[l]["lw"] for l in range(_L)])
    EB = jnp.stack(
        [params["be"] @ layers[l]["lw"] + layers[l]["lb"] for l in range(_L)]
    ).reshape(_L, 1, _H)

    h = _embed(x, params["Wn"], params["bn"])
    e3 = _edge_e(edge_attr, EW, EB)

    for l in range(_L):
        p = layers[l]
        agg = _sc_edge(h, e3[l], src4, dst4)
        scale = (1.0 + p["eps"]).reshape(1)
        h = _mlp(h, agg, scale, p["W1"], p["b1"], p["W2"], p["b2"])

    return _pool(h, batch.reshape(_N, 1), params["Wm1"], params["bm1"],
                 params["Wm2"], params["bm2"])


# speedup vs baseline: 2.8653x; 1.2260x over previous
import functools

import jax
import jax.numpy as jnp
from jax import lax
from jax.experimental import pallas as pl
from jax.experimental.pallas import tpu as pltpu
from jax.experimental.pallas import tpu_sc as plsc

_N = 10000
_E = 320000
_G = 64
_H = 128
_L = 3

_NC = 2
_NS = 16
_EPW = _E // (_NC * _NS)
_C = 16
_NCH = _EPW // _C
_CPG = 125
_GRP = _NCH // _CPG
_RPS = 624


def _embed_body(x_ref, w_ref, b_ref, o_ref):
    o_ref[...] = (
        jnp.dot(x_ref[...], w_ref[...], preferred_element_type=jnp.float32)
        + b_ref[...]
    )


def _embed(x, W, b):
    return pl.pallas_call(
        _embed_body,
        out_shape=jax.ShapeDtypeStruct((_N, _H), jnp.float32),
    )(x, W, b.reshape(1, _H))


_RB = 4000


def _edge_e_body(ea_ref, w_ref, b_ref, o_ref):
    o_ref[0] = (
        jnp.dot(ea_ref[...], w_ref[0], preferred_element_type=jnp.float32)
        + b_ref[0]
    )


def _edge_e(edge_attr, EW, EB):
    return pl.pallas_call(
        _edge_e_body,
        grid=(_L, _E // _RB),
        in_specs=[
            pl.BlockSpec((_RB, 6), lambda l, r: (r, 0)),
            pl.BlockSpec((1, 6, _H), lambda l, r: (l, 0, 0)),
            pl.BlockSpec((1, 1, _H), lambda l, r: (l, 0, 0)),
        ],
        out_specs=pl.BlockSpec((1, _RB, _H), lambda l, r: (l, r, 0)),
        out_shape=jax.ShapeDtypeStruct((_L, _E, _H), jnp.float32),
    )(edge_attr, EW, EB)


_MB = 2000


def _mlp_body(s_ref, h_ref, a_ref, w1_ref, b1_ref, w2_ref, b2_ref, o_ref):
    z = s_ref[0] * h_ref[...] + a_ref[0] + a_ref[1]
    z = jnp.maximum(
        jnp.dot(z, w1_ref[...], preferred_element_type=jnp.float32) + b1_ref[...],
        0.0,
    )
    z = jnp.dot(z, w2_ref[...], preferred_element_type=jnp.float32) + b2_ref[...]
    o_ref[...] = jnp.maximum(z, 0.0)


def _mlp(h, agg, scale, W1, b1, W2, b2):
    return pl.pallas_call(
        _mlp_body,
        grid=(_N // _MB,),
        in_specs=[
            pl.BlockSpec(memory_space=pltpu.SMEM),
            pl.BlockSpec((_MB, _H), lambda r: (r, 0)),
            pl.BlockSpec((_NC, _MB, _H), lambda r: (0, r, 0)),
            pl.BlockSpec((_H, 2 * _H), lambda r: (0, 0)),
            pl.BlockSpec((1, 2 * _H), lambda r: (0, 0)),
            pl.BlockSpec((2 * _H, _H), lambda r: (0, 0)),
            pl.BlockSpec((1, _H), lambda r: (0, 0)),
        ],
        out_specs=pl.BlockSpec((_MB, _H), lambda r: (r, 0)),
        out_shape=jax.ShapeDtypeStruct((_N, _H), jnp.float32),
    )(scale, h, agg, W1, b1.reshape(1, 2 * _H), W2, b2.reshape(1, _H))


def _pool_body(h_ref, bat_ref, w1_ref, b1_ref, w2_ref, b2_ref, o_ref):
    ids = bat_ref[...]
    oh = (ids == lax.broadcasted_iota(jnp.int32, (1, _G), 1)).astype(jnp.float32)
    summed = lax.dot_general(
        oh, h_ref[...], (((0,), (0,)), ((), ())),
        preferred_element_type=jnp.float32,
    )
    counts = lax.dot_general(
        oh, jnp.ones((_N, 1), jnp.float32), (((0,), (0,)), ((), ())),
        preferred_element_type=jnp.float32,
    )
    g = summed / jnp.clip(counts, 1.0, None)
    g = jnp.maximum(
        jnp.dot(g, w1_ref[...], preferred_element_type=jnp.float32) + b1_ref[...],
        0.0,
    )
    o_ref[...] = (
        jnp.dot(g, w2_ref[...], preferred_element_type=jnp.float32) + b2_ref[...]
    )


def _pool(h, batch_col, Wm1, bm1, Wm2, bm2):
    return pl.pallas_call(
        _pool_body,
        out_shape=jax.ShapeDtypeStruct((_G, 1), jnp.float32),
    )(h, batch_col, Wm1, bm1.reshape(1, _H // 2), Wm2, bm2.reshape(1, 1))


def _sc_edge(h, e3, idx4, zeros_nh, l):
    mesh = plsc.VectorSubcoreMesh(core_axis_name="c", subcore_axis_name="s")

    @functools.partial(
        pl.kernel,
        out_type=jax.ShapeDtypeStruct((_NC, _N, _H), jnp.float32),
        mesh=mesh,
        scratch_types=[
            pltpu.VMEM((_CPG, _C), jnp.int32),
            pltpu.VMEM((_C, _H), jnp.float32),
            pltpu.VMEM((_C, _H), jnp.float32),
            pltpu.VMEM((_C, _H), jnp.float32),
            pltpu.VMEM((_C, _H), jnp.float32),
            pltpu.VMEM_SHARED((_N, _H), jnp.float32),
            pltpu.SemaphoreType.DMA,
            pltpu.SemaphoreType.DMA,
            pltpu.SemaphoreType.DMA,
            pltpu.SemaphoreType.DMA,
        ],
    )
    def body(h_hbm, e_hbm, idx_hbm, z_hbm, out_hbm,
             idxv, hv0, hv1, ev0, ev1, aggr, g0, g1, s0, s1):
        cid = lax.axis_index("c")
        sid = lax.axis_index("s")

        pltpu.sync_copy(
            z_hbm.at[pl.ds(sid * _RPS, _RPS)],
            aggr.at[pl.ds(sid * _RPS, _RPS)],
        )

        @pl.when(sid == _NS - 1)
        def _zero_tail():
            pltpu.sync_copy(
                z_hbm.at[pl.ds(_NS * _RPS, 16)],
                aggr.at[pl.ds(_NS * _RPS, 16)],
            )

        plsc.subcore_barrier()

        ebase = (cid * _NS + sid) * _EPW

        def wait_g(hv, ev, sem):
            pltpu.make_async_copy(h_hbm.at[pl.ds(0, _C)], hv, sem).wait()
            pltpu.make_async_copy(e_hbm.at[l, pl.ds(0, _C)], ev, sem).wait()

        def compute(hv, ev):
            def row(i, carry):
                for j in range(_H // 16):
                    s = pl.ds(j * 16, 16)
                    hv[i, s] = jnp.maximum(hv[i, s] + ev[i, s], 0.0)
                return carry

            lax.fori_loop(0, _C, row, 0)

        def issue_s(k, hv, sem):
            w = idxv[k]
            didx = lax.shift_right_logical(w, 16)
            pltpu.async_copy(hv, aggr.at[didx], sem, add=True)

        def wait_s(hv, sem):
            pltpu.make_async_copy(hv, aggr.at[pl.ds(0, _C)], sem).wait()

        def group(g, carry):
            pltpu.sync_copy(idx_hbm.at[cid, sid, g], idxv)
            gbase = g * _CPG

            def issue_g(k, hv, ev, sem):
                w = idxv[k]
                sidx = jnp.bitwise_and(w, 0xFFFF)
                pltpu.async_copy(h_hbm.at[sidx], hv, sem)
                pltpu.async_copy(
                    e_hbm.at[l, pl.ds(ebase + (gbase + k) * _C, _C)], ev, sem)

            issue_g(0, hv0, ev0, g0)
            issue_g(1, hv1, ev1, g1)

            def pair(i, c2):
                a = 2 * i
                wait_g(hv0, ev0, g0)
                compute(hv0, ev0)
                issue_s(a, hv0, s0)
                wait_g(hv1, ev1, g1)
                compute(hv1, ev1)
                issue_s(a + 1, hv1, s1)
                wait_s(hv0, s0)

                @pl.when(a + 2 < _CPG)
                def _next0():
                    issue_g(a + 2, hv0, ev0, g0)

                wait_s(hv1, s1)

                @pl.when(a + 3 < _CPG)
                def _next1():
                    issue_g(a + 3, hv1, ev1, g1)

                return c2

            lax.fori_loop(0, _CPG // 2, pair, 0)

            wait_g(hv0, ev0, g0)
            compute(hv0, ev0)
            issue_s(_CPG - 1, hv0, s0)
            wait_s(hv0, s0)
            return carry

        lax.fori_loop(0, _GRP, group, 0)

        plsc.subcore_barrier()

        for k in range(3):
            pltpu.sync_copy(
                aggr.at[pl.ds(sid * _RPS + k * 208, 208)],
                out_hbm.at[cid, pl.ds(sid * _RPS + k * 208, 208)],
            )

        @pl.when(sid == _NS - 1)
        def _out_tail():
            pltpu.sync_copy(
                aggr.at[pl.ds(_NS * _RPS, 16)],
                out_hbm.at[cid, pl.ds(_NS * _RPS, 16)],
            )

    return body(h, e3, idx4, zeros_nh)


def kernel(x, edge_attr, edge_index, batch, params):
    packed = edge_index[0] + (edge_index[1] << 16)
    idx4 = packed.reshape(_NC, _NS, _GRP, _CPG, _C)
    zeros_nh = jnp.zeros((_N, _H), jnp.float32)

    layers = params["layers"]
    EW = jnp.stack([params["We"] @ layers[l]["lw"] for l in range(_L)])
    EB = jnp.stack(
        [params["be"] @ layers[l]["lw"] + layers[l]["lb"] for l in range(_L)]
    ).reshape(_L, 1, _H)

    h = _embed(x, params["Wn"], params["bn"])
    e3 = _edge_e(edge_attr, EW, EB)

    for l in range(_L):
        p = layers[l]
        agg = _sc_edge(h, e3, idx4, zeros_nh, l)
        scale = (1.0 + p["eps"]).reshape(1)
        h = _mlp(h, agg, scale, p["W1"], p["b1"], p["W2"], p["b2"])

    return _pool(h, batch.reshape(_N, 1), params["Wm1"], params["bm1"],
                 params["Wm2"], params["bm2"])
